# Initial kernel scaffold; baseline (speedup 1.0000x reference)
import jax
import jax.numpy as jnp
from jax.experimental import pallas as pl


def kernel(x, edge_index, batch, W1, b1, g1, be1, m1, v1, W2, b2, g2, be2, m2, v2, W3, b3):
    raise NotImplementedError("write your pallas kernel here")


# speedup vs baseline: 2.4642x; 2.4642x over previous
import functools

import jax
import jax.numpy as jnp
from jax.experimental import pallas as pl

_N = 50000
_GRAPHS = 256
_EPS = 1e-5
_BLK = 2000


def _mm_scale_kernel(h_ref, w_ref, dinv_ref, o_ref):
    o_ref[...] = jnp.dot(h_ref[...], w_ref[...],
                         preferred_element_type=jnp.float32) * dinv_ref[...]


def _post_kernel(s_ref, p_ref, dinv_ref, sc_ref, bi_ref, o_ref, *, relu):
    c = (s_ref[...] + p_ref[...]) * dinv_ref[...]
    y = c * sc_ref[...] + bi_ref[...]
    if relu:
        y = jnp.maximum(y, 0.0)
    o_ref[...] = y


def _mm_scale(h, w, dinv):
    n, d_in = h.shape
    d_out = w.shape[1]
    return pl.pallas_call(
        _mm_scale_kernel,
        grid=(n // _BLK,),
        in_specs=[
            pl.BlockSpec((_BLK, d_in), lambda i: (i, 0)),
            pl.BlockSpec((d_in, d_out), lambda i: (0, 0)),
            pl.BlockSpec((_BLK, 1), lambda i: (i, 0)),
        ],
        out_specs=pl.BlockSpec((_BLK, d_out), lambda i: (i, 0)),
        out_shape=jax.ShapeDtypeStruct((n, d_out), jnp.float32),
    )(h, w, dinv)


def _post(s, p, dinv, scale, bias, relu):
    n, d = s.shape
    return pl.pallas_call(
        functools.partial(_post_kernel, relu=relu),
        grid=(n // _BLK,),
        in_specs=[
            pl.BlockSpec((_BLK, d), lambda i: (i, 0)),
            pl.BlockSpec((_BLK, d), lambda i: (i, 0)),
            pl.BlockSpec((_BLK, 1), lambda i: (i, 0)),
            pl.BlockSpec((1, d), lambda i: (0, 0)),
            pl.BlockSpec((1, d), lambda i: (0, 0)),
        ],
        out_specs=pl.BlockSpec((_BLK, d), lambda i: (i, 0)),
        out_shape=jax.ShapeDtypeStruct((n, d), jnp.float32),
    )(s, p, dinv, scale.reshape(1, d), bias.reshape(1, d))


def _layer(h, src, dst, dinv, w, scale, bias, relu):
    p = _mm_scale(h, w, dinv)
    s = jnp.zeros_like(p).at[dst].add(p[src])
    return _post(s, p, dinv, scale, bias, relu)


def kernel(x, edge_index, batch, W1, b1, g1, be1, m1, v1,
           W2, b2, g2, be2, m2, v2, W3, b3):
    src, dst = edge_index[0], edge_index[1]
    deg = 1.0 + jnp.zeros((_N,), jnp.float32).at[dst].add(1.0)
    dinv = jax.lax.rsqrt(deg).reshape(_N, 1)

    k1 = jax.lax.rsqrt(v1 + _EPS) * g1
    k2 = jax.lax.rsqrt(v2 + _EPS) * g2
    h = _layer(x, src, dst, dinv, W1, k1, (b1 - m1) * k1 + be1, True)
    h = _layer(h, src, dst, dinv, W2, k2, (b2 - m2) * k2 + be2, True)
    h = _layer(h, src, dst, dinv, W3, jnp.ones_like(b3), b3, False)
    return jax.ops.segment_max(h, batch, num_segments=_GRAPHS)
